# baseline (speedup 1.0000x reference)
import functools

import jax
import jax.numpy as jnp
from jax import lax
from jax.experimental import pallas as pl
from jax.experimental.pallas import tpu as pltpu
from jax.experimental.pallas import tpu_sc as plsc

_B, _N, _K, _D = 2, 2048, 30, 32
_DE = 95
_H = 4 * _D
_TW = 128
_BN = 256
_EB = _BN * _K
_CH = 256


def _sc_gather(table, idx):
    E = idx.shape[0]
    info = plsc.get_sparse_core_info()
    nw = info.num_cores * info.num_subcores
    per_w = E // nw
    n_ch = per_w // _CH
    mesh = plsc.VectorSubcoreMesh(core_axis_name="c", subcore_axis_name="s")

    @functools.partial(
        pl.kernel,
        mesh=mesh,
        out_type=jax.ShapeDtypeStruct((E, _TW), jnp.float32),
        scratch_types=[
            pltpu.VMEM((_CH,), jnp.int32),
            pltpu.VMEM((_CH, _TW), jnp.float32),
            pltpu.SemaphoreType.DMA,
        ],
    )
    def gk(tab_hbm, idx_hbm, out_hbm, idx_v, rows_v, sem):
        wid = lax.axis_index("s") * info.num_cores + lax.axis_index("c")
        base0 = wid * per_w

        def body(c, carry):
            base = base0 + c * _CH
            pltpu.sync_copy(idx_hbm.at[pl.ds(base, _CH)], idx_v)
            pltpu.async_copy(tab_hbm.at[idx_v], rows_v, sem).wait()
            pltpu.sync_copy(rows_v, out_hbm.at[pl.ds(base, _CH)])
            return carry

        lax.fori_loop(0, n_ch, body, 0)

    return gk(table, idx)


def _gelu(x):
    return x * 0.5 * (1.0 + lax.erf(x * 0.7071067811865476))


def _tc_body(tab_ref, gat_ref, edg_ref,
             we1f_ref, we1j_ref, we1fe_ref, we1e_ref, be1_ref,
             we2_ref, be2_ref, wg_ref, bg_ref,
             wc1_ref, bc1_ref, wc2_ref, bc2_ref,
             wn1a_ref, wn1b_ref, bn1_ref, wn2_ref, bn2_ref,
             lng_ref, lnb_ref, cs_ref,
             node_ref, space_ref):
    f32 = jnp.float32
    tab = tab_ref[0]
    feats = tab[:, 0:_D]
    space = tab[:, _D:_D + 3]
    gat = gat_ref[0]
    space_j = gat[:, _D:_D + 3]
    edg = edg_ref[0]

    h_i = jnp.dot(feats, we1f_ref[...], preferred_element_type=f32)
    h = jnp.broadcast_to(h_i[:, None, :], (_BN, _K, _H)).reshape(_EB, _H)
    h = h + jnp.dot(gat, we1j_ref[...], preferred_element_type=f32)
    h = h + jnp.dot(edg, we1e_ref[...], preferred_element_type=f32)

    vec = space_j - jnp.broadcast_to(space[:, None, :], (_BN, _K, 3)).reshape(_EB, 3)
    vec_t = jnp.transpose(vec)
    eu_t = jnp.sum(vec_t * vec_t, axis=0, keepdims=True)
    th = eu_t * 0.125
    s3 = jnp.sin(th)
    c3 = jnp.cos(th)
    s2 = 2.0 * s3 * c3
    c2 = 1.0 - 2.0 * s3 * s3
    s1 = 2.0 * s2 * c2
    c1 = 1.0 - 2.0 * s2 * s2
    s0 = 2.0 * s1 * c1
    c0 = 1.0 - 2.0 * s1 * s1
    ft = jnp.concatenate([s0, s1, s2, s3, c0, c1, c2, c3, eu_t], axis=0)
    fe = jnp.transpose(ft)
    h = h + jnp.dot(fe, we1fe_ref[...], preferred_element_type=f32)
    h = _gelu(h + be1_ref[...])

    m = jnp.dot(h, we2_ref[...], preferred_element_type=f32) + be2_ref[...]
    g = jax.nn.sigmoid(jnp.sum(m * wg_ref[...], axis=-1, keepdims=True) + bg_ref[0, 0])
    m = m * g

    hc = _gelu(jnp.dot(m, wc1_ref[...], preferred_element_type=f32) + bc1_ref[...])
    vw = jnp.sum(hc * wc2_ref[...], axis=-1, keepdims=True) + bc2_ref[0, 0]
    vw_t = jnp.transpose(vw)
    rn_t = vw_t * cs_ref[0, 0] / jnp.maximum(jnp.sqrt(eu_t), 1e-8)
    contrib = jnp.transpose(vec_t * rn_t)
    space_ref[0] = jnp.sum(contrib.reshape(_BN, _K, 3), axis=1) + space

    m_i = jnp.sum(m.reshape(_BN, _K, _D), axis=1)
    mu = jnp.mean(feats, axis=-1, keepdims=True)
    var = jnp.mean((feats - mu) ** 2, axis=-1, keepdims=True)
    normed = (feats - mu) / jnp.sqrt(var + 1e-5) * lng_ref[...] + lnb_ref[...]
    hn = _gelu(jnp.dot(normed, wn1a_ref[...], preferred_element_type=f32)
               + jnp.dot(m_i, wn1b_ref[...], preferred_element_type=f32)
               + bn1_ref[...])
    node_ref[0] = (jnp.dot(hn, wn2_ref[...], preferred_element_type=f32)
                   + bn2_ref[...] + feats)


def kernel(feats, space, edges, E_idx, W_e1, b_e1, W_e2, b_e2, W_g, b_g,
           W_c1, b_c1, W_c2, b_c2, W_n1, b_n1, W_n2, b_n2, ln_g, ln_b,
           coors_scale):
    f32 = jnp.float32
    feats = feats.astype(f32)
    space = space.astype(f32)
    table = jnp.concatenate(
        [feats, space, jnp.zeros((_B, _N, _TW - _D - 3), f32)], axis=-1)
    table_flat = table.reshape(_B * _N, _TW)
    idx = (E_idx.astype(jnp.int32)
           + (jnp.arange(_B, dtype=jnp.int32) * _N)[:, None, None])
    gathered = _sc_gather(table_flat, idx.reshape(_B * _N * _K))
    gathered = gathered.reshape(_B, _N * _K, _TW)
    edges_flat = edges.astype(f32).reshape(_B, _N * _K, _DE)

    W_e1 = W_e1.astype(f32)
    we1f = W_e1[0:_D]
    we1j = jnp.concatenate(
        [W_e1[_D:2 * _D], jnp.zeros((_TW - _D, _H), f32)], axis=0)
    we1fe = W_e1[2 * _D:2 * _D + 9]
    we1e = W_e1[2 * _D + 9:]
    row = lambda v: v.astype(f32).reshape(1, -1)
    wn1a = W_n1.astype(f32)[0:_D]
    wn1b = W_n1.astype(f32)[_D:]

    nb = _N // _BN
    grid = (_B, nb)
    blk = lambda shp, imap: pl.BlockSpec(shp, imap)
    full = lambda a: pl.BlockSpec(a.shape, lambda b, i: (0,) * a.ndim)
    edge_map = lambda b, i: (b, i, 0)

    weights = [we1f, we1j, we1fe, we1e, row(b_e1),
               W_e2.astype(f32), row(b_e2), row(W_g), row(b_g).reshape(1, 1),
               W_c1.astype(f32), row(b_c1), row(W_c2), row(b_c2).reshape(1, 1),
               wn1a, wn1b, row(b_n1), W_n2.astype(f32), row(b_n2),
               row(ln_g), row(ln_b), row(coors_scale).reshape(1, 1)]

    node_out, space_out = pl.pallas_call(
        _tc_body,
        grid=grid,
        in_specs=[
            blk((1, _BN, _TW), edge_map),
            blk((1, _EB, _TW), edge_map),
            blk((1, _EB, _DE), edge_map),
        ] + [full(w) for w in weights],
        out_specs=[
            blk((1, _BN, _D), edge_map),
            blk((1, _BN, 3), edge_map),
        ],
        out_shape=[
            jax.ShapeDtypeStruct((_B, _N, _D), f32),
            jax.ShapeDtypeStruct((_B, _N, 3), f32),
        ],
        compiler_params=pltpu.CompilerParams(
            dimension_semantics=("parallel", "parallel")),
    )(table, gathered, edges_flat, *weights)

    return node_out, space_out


# speedup vs baseline: 1.0379x; 1.0379x over previous
import functools

import jax
import jax.numpy as jnp
from jax import lax
from jax.experimental import pallas as pl
from jax.experimental.pallas import tpu as pltpu
from jax.experimental.pallas import tpu_sc as plsc

_B, _N, _K, _D = 2, 2048, 30, 32
_DE = 95
_H = 4 * _D
_TW = 128
_BN = 256
_EB = _BN * _K
_CH = 256


def _sc_gather(table, idx):
    E = idx.shape[0]
    info = plsc.get_sparse_core_info()
    nw = info.num_cores * info.num_subcores
    per_w = E // nw
    n_ch = per_w // _CH
    mesh = plsc.VectorSubcoreMesh(core_axis_name="c", subcore_axis_name="s")

    @functools.partial(
        pl.kernel,
        mesh=mesh,
        out_type=jax.ShapeDtypeStruct((E, _TW), jnp.float32),
        scratch_types=[
            pltpu.VMEM((_CH,), jnp.int32),
            pltpu.VMEM((_CH, _TW), jnp.float32),
            pltpu.SemaphoreType.DMA,
        ],
    )
    def gk(tab_hbm, idx_hbm, out_hbm, idx_v, rows_v, sem):
        wid = lax.axis_index("s") * info.num_cores + lax.axis_index("c")
        base0 = wid * per_w

        def body(c, carry):
            base = base0 + c * _CH
            pltpu.sync_copy(idx_hbm.at[pl.ds(base, _CH)], idx_v)
            pltpu.async_copy(tab_hbm.at[idx_v], rows_v, sem).wait()
            pltpu.sync_copy(rows_v, out_hbm.at[pl.ds(base, _CH)])
            return carry

        lax.fori_loop(0, n_ch, body, 0)

    return gk(table, idx)


def _gelu(x):
    return x * 0.5 * (1.0 + lax.erf(x * 0.7071067811865476))


def _tc_body(tab_ref, gat_ref, edg_ref,
             we1f_ref, we1j_ref, we1fe_ref, we1e_ref, be1_ref,
             we2_ref, be2_ref, wg_ref, bg_ref,
             wc1_ref, bc1_ref, wc2_ref, bc2_ref,
             wn1a_ref, wn1b_ref, bn1_ref, wn2_ref, bn2_ref,
             lng_ref, lnb_ref, cs_ref,
             node_ref, space_ref):
    f32 = jnp.float32
    tab = tab_ref[0]
    feats = tab[:, 0:_D]
    space = tab[:, _D:_D + 3]
    gat = gat_ref[0]
    space_j = gat[:, _D:_D + 3]
    edg = edg_ref[0].reshape(_EB, _DE)

    h_i = jnp.dot(feats, we1f_ref[...], preferred_element_type=f32)
    h = jnp.broadcast_to(h_i[:, None, :], (_BN, _K, _H)).reshape(_EB, _H)
    h = h + jnp.dot(gat, we1j_ref[...], preferred_element_type=f32)
    h = h + jnp.dot(edg, we1e_ref[...], preferred_element_type=f32)

    vec = space_j - jnp.broadcast_to(space[:, None, :], (_BN, _K, 3)).reshape(_EB, 3)
    vec_t = jnp.transpose(vec)
    eu_t = jnp.sum(vec_t * vec_t, axis=0, keepdims=True)
    th = eu_t * 0.125
    s3 = jnp.sin(th)
    c3 = jnp.cos(th)
    s2 = 2.0 * s3 * c3
    c2 = 1.0 - 2.0 * s3 * s3
    s1 = 2.0 * s2 * c2
    c1 = 1.0 - 2.0 * s2 * s2
    s0 = 2.0 * s1 * c1
    c0 = 1.0 - 2.0 * s1 * s1
    ft = jnp.concatenate([s0, s1, s2, s3, c0, c1, c2, c3, eu_t], axis=0)
    fe = jnp.transpose(ft)
    h = h + jnp.dot(fe, we1fe_ref[...], preferred_element_type=f32)
    h = _gelu(h + be1_ref[...])

    m = jnp.dot(h, we2_ref[...], preferred_element_type=f32) + be2_ref[...]
    g = jax.nn.sigmoid(jnp.sum(m * wg_ref[...], axis=-1, keepdims=True) + bg_ref[0, 0])
    m = m * g

    hc = _gelu(jnp.dot(m, wc1_ref[...], preferred_element_type=f32) + bc1_ref[...])
    vw = jnp.sum(hc * wc2_ref[...], axis=-1, keepdims=True) + bc2_ref[0, 0]
    vw_t = jnp.transpose(vw)
    rn_t = vw_t * cs_ref[0, 0] / jnp.maximum(jnp.sqrt(eu_t), 1e-8)
    contrib = jnp.transpose(vec_t * rn_t)
    space_ref[0] = jnp.sum(contrib.reshape(_BN, _K, 3), axis=1) + space

    m_i = jnp.sum(m.reshape(_BN, _K, _D), axis=1)
    mu = jnp.mean(feats, axis=-1, keepdims=True)
    var = jnp.mean((feats - mu) ** 2, axis=-1, keepdims=True)
    normed = (feats - mu) / jnp.sqrt(var + 1e-5) * lng_ref[...] + lnb_ref[...]
    hn = _gelu(jnp.dot(normed, wn1a_ref[...], preferred_element_type=f32)
               + jnp.dot(m_i, wn1b_ref[...], preferred_element_type=f32)
               + bn1_ref[...])
    node_ref[0] = (jnp.dot(hn, wn2_ref[...], preferred_element_type=f32)
                   + bn2_ref[...] + feats)


def kernel(feats, space, edges, E_idx, W_e1, b_e1, W_e2, b_e2, W_g, b_g,
           W_c1, b_c1, W_c2, b_c2, W_n1, b_n1, W_n2, b_n2, ln_g, ln_b,
           coors_scale):
    f32 = jnp.float32
    feats = feats.astype(f32)
    space = space.astype(f32)
    table = jnp.concatenate(
        [feats, space, jnp.zeros((_B, _N, _TW - _D - 3), f32)], axis=-1)
    table_flat = table.reshape(_B * _N, _TW)
    idx = (E_idx.astype(jnp.int32)
           + (jnp.arange(_B, dtype=jnp.int32) * _N)[:, None, None])
    gathered = _sc_gather(table_flat, idx.reshape(_B * _N * _K))
    gathered = gathered.reshape(_B, _N * _K, _TW)
    edges4 = edges.astype(f32)

    W_e1 = W_e1.astype(f32)
    we1f = W_e1[0:_D]
    we1j = jnp.concatenate(
        [W_e1[_D:2 * _D], jnp.zeros((_TW - _D, _H), f32)], axis=0)
    we1fe = W_e1[2 * _D:2 * _D + 9]
    we1e = W_e1[2 * _D + 9:]
    row = lambda v: v.astype(f32).reshape(1, -1)
    wn1a = W_n1.astype(f32)[0:_D]
    wn1b = W_n1.astype(f32)[_D:]

    nb = _N // _BN
    grid = (_B, nb)
    blk = lambda shp, imap: pl.BlockSpec(shp, imap)
    full = lambda a: pl.BlockSpec(a.shape, lambda b, i: (0,) * a.ndim)
    edge_map = lambda b, i: (b, i, 0)

    weights = [we1f, we1j, we1fe, we1e, row(b_e1),
               W_e2.astype(f32), row(b_e2), row(W_g), row(b_g).reshape(1, 1),
               W_c1.astype(f32), row(b_c1), row(W_c2), row(b_c2).reshape(1, 1),
               wn1a, wn1b, row(b_n1), W_n2.astype(f32), row(b_n2),
               row(ln_g), row(ln_b), row(coors_scale).reshape(1, 1)]

    node_out, space_out = pl.pallas_call(
        _tc_body,
        grid=grid,
        in_specs=[
            blk((1, _BN, _TW), edge_map),
            blk((1, _EB, _TW), edge_map),
            pl.BlockSpec((1, _BN, _K, _DE), lambda b, i: (b, i, 0, 0)),
        ] + [full(w) for w in weights],
        out_specs=[
            blk((1, _BN, _D), edge_map),
            blk((1, _BN, 3), edge_map),
        ],
        out_shape=[
            jax.ShapeDtypeStruct((_B, _N, _D), f32),
            jax.ShapeDtypeStruct((_B, _N, 3), f32),
        ],
        compiler_params=pltpu.CompilerParams(
            dimension_semantics=("parallel", "parallel")),
    )(table, gathered, edges4, *weights)

    return node_out, space_out


# speedup vs baseline: 1.6106x; 1.5518x over previous
import functools

import jax
import jax.numpy as jnp
from jax import lax
from jax.experimental import pallas as pl
from jax.experimental.pallas import tpu as pltpu
from jax.experimental.pallas import tpu_sc as plsc

_B, _N, _K, _D = 2, 2048, 30, 32
_DE = 95
_H = 4 * _D
_TW = 128
_BN = 128
_EB = _B * _K * _BN
_NBL = _B * _BN
_CH = 256


def _sc_gather(table, idx):
    E = idx.shape[0]
    info = plsc.get_sparse_core_info()
    nw = info.num_cores * info.num_subcores
    per_w = E // nw
    n_ch = per_w // _CH
    mesh = plsc.VectorSubcoreMesh(core_axis_name="c", subcore_axis_name="s")

    @functools.partial(
        pl.kernel,
        mesh=mesh,
        out_type=jax.ShapeDtypeStruct((E, _TW), jnp.float32),
        scratch_types=[
            pltpu.VMEM((_CH,), jnp.int32),
            pltpu.VMEM((_CH, _TW), jnp.float32),
            pltpu.SemaphoreType.DMA,
        ],
    )
    def gk(tab_hbm, idx_hbm, out_hbm, idx_v, rows_v, sem):
        wid = lax.axis_index("s") * info.num_cores + lax.axis_index("c")
        base0 = wid * per_w

        def body(c, carry):
            base = base0 + c * _CH
            pltpu.sync_copy(idx_hbm.at[pl.ds(base, _CH)], idx_v)
            pltpu.async_copy(tab_hbm.at[idx_v], rows_v, sem).wait()
            pltpu.sync_copy(rows_v, out_hbm.at[pl.ds(base, _CH)])
            return carry

        lax.fori_loop(0, n_ch, body, 0)

    return gk(table, idx)


def _gelu(x):
    return x * 0.5 * (1.0 + lax.erf(x * 0.7071067811865476))


def _tc_body(tab_ref, gat_ref, edg_ref,
             we1f_ref, we1j_ref, we1fe_ref, we1e_ref, be1_ref,
             we2_ref, be2_ref, wg_ref, bg_ref,
             wc1_ref, bc1_ref, wc2_ref, bc2_ref,
             wn1a_ref, wn1b_ref, bn1_ref, wn2_ref, bn2_ref,
             lng_ref, lnb_ref, cs_ref,
             node_ref, space_ref):
    f32 = jnp.float32
    tab = tab_ref[...]
    feats = tab[:, :, 0:_D].reshape(_NBL, _D)
    space = tab[:, :, _D:_D + 3]
    gat = gat_ref[...].reshape(_EB, _TW)
    space_j = gat[:, _D:_D + 3]

    h_i = jnp.dot(feats, we1f_ref[...], preferred_element_type=f32)
    h = jnp.broadcast_to(h_i.reshape(_B, 1, _BN, _H),
                         (_B, _K, _BN, _H)).reshape(_EB, _H)
    h = h + jnp.dot(gat, we1j_ref[...], preferred_element_type=f32)
    he = []
    for b in range(_B):
        for k in range(_K):
            he.append(lax.dot_general(
                edg_ref[k, :, b, :], we1e_ref[...],
                (((0,), (0,)), ((), ())), preferred_element_type=f32))
    h = h + jnp.concatenate(he, axis=0)

    vec = space_j - jnp.broadcast_to(space[:, None], (_B, _K, _BN, 3)).reshape(_EB, 3)
    vec_t = jnp.transpose(vec)
    eu_t = jnp.sum(vec_t * vec_t, axis=0, keepdims=True)
    th = eu_t * 0.125
    s3 = jnp.sin(th)
    c3 = jnp.cos(th)
    s2 = 2.0 * s3 * c3
    c2 = 1.0 - 2.0 * s3 * s3
    s1 = 2.0 * s2 * c2
    c1 = 1.0 - 2.0 * s2 * s2
    s0 = 2.0 * s1 * c1
    c0 = 1.0 - 2.0 * s1 * s1
    ft = jnp.concatenate([s0, s1, s2, s3, c0, c1, c2, c3, eu_t], axis=0)
    fe = jnp.transpose(ft)
    h = h + jnp.dot(fe, we1fe_ref[...], preferred_element_type=f32)
    h = _gelu(h + be1_ref[...])

    m = jnp.dot(h, we2_ref[...], preferred_element_type=f32) + be2_ref[...]
    g = jax.nn.sigmoid(jnp.sum(m * wg_ref[...], axis=-1, keepdims=True) + bg_ref[0, 0])
    m = m * g

    hc = _gelu(jnp.dot(m, wc1_ref[...], preferred_element_type=f32) + bc1_ref[...])
    vw = jnp.sum(hc * wc2_ref[...], axis=-1, keepdims=True) + bc2_ref[0, 0]
    vw_t = jnp.transpose(vw)
    rn_t = vw_t * cs_ref[0, 0] / jnp.maximum(jnp.sqrt(eu_t), 1e-8)
    contrib = jnp.transpose(vec_t * rn_t)
    space_ref[...] = (jnp.sum(contrib.reshape(_B, _K, _BN, 3), axis=1)
                      + space)

    m_i = jnp.sum(m.reshape(_B, _K, _BN, _D), axis=1).reshape(_NBL, _D)
    mu = jnp.mean(feats, axis=-1, keepdims=True)
    var = jnp.mean((feats - mu) ** 2, axis=-1, keepdims=True)
    normed = (feats - mu) / jnp.sqrt(var + 1e-5) * lng_ref[...] + lnb_ref[...]
    hn = _gelu(jnp.dot(normed, wn1a_ref[...], preferred_element_type=f32)
               + jnp.dot(m_i, wn1b_ref[...], preferred_element_type=f32)
               + bn1_ref[...])
    node = (jnp.dot(hn, wn2_ref[...], preferred_element_type=f32)
            + bn2_ref[...] + feats)
    node_ref[...] = node.reshape(_B, _BN, _D)


def kernel(feats, space, edges, E_idx, W_e1, b_e1, W_e2, b_e2, W_g, b_g,
           W_c1, b_c1, W_c2, b_c2, W_n1, b_n1, W_n2, b_n2, ln_g, ln_b,
           coors_scale):
    f32 = jnp.float32
    feats = feats.astype(f32)
    space = space.astype(f32)
    table = jnp.concatenate(
        [feats, space, jnp.zeros((_B, _N, _TW - _D - 3), f32)], axis=-1)
    table_flat = table.reshape(_B * _N, _TW)
    idx = (jnp.transpose(E_idx.astype(jnp.int32), (0, 2, 1))
           + (jnp.arange(_B, dtype=jnp.int32) * _N)[:, None, None])
    gathered = _sc_gather(table_flat, idx.reshape(_B * _K * _N))
    gathered = gathered.reshape(_B, _K, _N, _TW)
    edges_t = jnp.transpose(edges.astype(f32), (2, 3, 0, 1))

    W_e1 = W_e1.astype(f32)
    we1f = W_e1[0:_D]
    we1j = jnp.concatenate(
        [W_e1[_D:2 * _D], jnp.zeros((_TW - _D, _H), f32)], axis=0)
    we1fe = W_e1[2 * _D:2 * _D + 9]
    we1e = W_e1[2 * _D + 9:]
    row = lambda v: v.astype(f32).reshape(1, -1)
    wn1a = W_n1.astype(f32)[0:_D]
    wn1b = W_n1.astype(f32)[_D:]

    nb = _N // _BN
    grid = (nb,)
    full = lambda a: pl.BlockSpec(a.shape, lambda i: (0,) * a.ndim)

    weights = [we1f, we1j, we1fe, we1e, row(b_e1),
               W_e2.astype(f32), row(b_e2), row(W_g), row(b_g).reshape(1, 1),
               W_c1.astype(f32), row(b_c1), row(W_c2), row(b_c2).reshape(1, 1),
               wn1a, wn1b, row(b_n1), W_n2.astype(f32), row(b_n2),
               row(ln_g), row(ln_b), row(coors_scale).reshape(1, 1)]

    node_out, space_out = pl.pallas_call(
        _tc_body,
        grid=grid,
        in_specs=[
            pl.BlockSpec((_B, _BN, _TW), lambda i: (0, i, 0)),
            pl.BlockSpec((_B, _K, _BN, _TW), lambda i: (0, 0, i, 0)),
            pl.BlockSpec((_K, _DE, _B, _BN), lambda i: (0, 0, 0, i)),
        ] + [full(w) for w in weights],
        out_specs=[
            pl.BlockSpec((_B, _BN, _D), lambda i: (0, i, 0)),
            pl.BlockSpec((_B, _BN, 3), lambda i: (0, i, 0)),
        ],
        out_shape=[
            jax.ShapeDtypeStruct((_B, _N, _D), f32),
            jax.ShapeDtypeStruct((_B, _N, 3), f32),
        ],
        compiler_params=pltpu.CompilerParams(
            dimension_semantics=("parallel",)),
    )(table, gathered, edges_t, *weights)

    return node_out, space_out


# speedup vs baseline: 1.7987x; 1.1168x over previous
import functools

import jax
import jax.numpy as jnp
from jax import lax
from jax.experimental import pallas as pl
from jax.experimental.pallas import tpu as pltpu
from jax.experimental.pallas import tpu_sc as plsc

_B, _N, _K, _D = 2, 2048, 30, 32
_DE = 95
_H = 4 * _D
_TW = 128
_BN = 128
_NS = 2
_EB = _B * _K * _BN
_NBL = _B * _BN
_CH = 256


def _pick_chunk(per_w):
    for ch in range(min(per_w, 256), 7, -8):
        if per_w % ch == 0:
            return ch
    return per_w


def _sc_gather(table, idx):
    E = idx.shape[0]
    info = plsc.get_sparse_core_info()
    nw = info.num_cores * info.num_subcores
    per_w = E // nw
    _CH = _pick_chunk(per_w)
    n_ch = per_w // _CH
    mesh = plsc.VectorSubcoreMesh(core_axis_name="c", subcore_axis_name="s")

    @functools.partial(
        pl.kernel,
        mesh=mesh,
        out_type=jax.ShapeDtypeStruct((E, _TW), jnp.float32),
        scratch_types=[
            pltpu.VMEM((_CH,), jnp.int32),
            pltpu.VMEM((_CH,), jnp.int32),
            pltpu.VMEM((_CH, _TW), jnp.float32),
            pltpu.VMEM((_CH, _TW), jnp.float32),
            pltpu.SemaphoreType.DMA,
            pltpu.SemaphoreType.DMA,
            pltpu.SemaphoreType.DMA,
        ],
    )
    def gk(tab_hbm, idx_hbm, out_hbm, idx0, idx1, rows0, rows1,
           gsem, osem0, osem1):
        wid = lax.axis_index("s") * info.num_cores + lax.axis_index("c")
        base0 = wid * per_w
        idxs, rows, osems = (idx0, idx1), (rows0, rows1), (osem0, osem1)
        out_pending = [None, None]
        pltpu.sync_copy(idx_hbm.at[pl.ds(base0, _CH)], idx0)
        hg = pltpu.async_copy(tab_hbm.at[idx0], rows0, gsem)
        for c in range(1, n_ch + 1):
            p, q = c % 2, (c - 1) % 2
            if c < n_ch:
                pltpu.sync_copy(idx_hbm.at[pl.ds(base0 + c * _CH, _CH)],
                                idxs[p])
            hg.wait()
            out_pending[q] = pltpu.async_copy(
                rows[q], out_hbm.at[pl.ds(base0 + (c - 1) * _CH, _CH)],
                osems[q])
            if c < n_ch:
                if out_pending[p] is not None:
                    out_pending[p].wait()
                    out_pending[p] = None
                hg = pltpu.async_copy(tab_hbm.at[idxs[p]], rows[p], gsem)
        for h in out_pending:
            if h is not None:
                h.wait()

    return gk(table, idx)


def _gelu(x):
    return x * 0.5 * (1.0 + lax.erf(x * 0.7071067811865476))


def _tc_body(tab_ref, gat_ref, edg_ref,
             we1f_ref, we1j_ref, we1fe_ref, we1e_ref, be1_ref,
             we2_ref, be2_ref, wg_ref, bg_ref,
             wc1_ref, bc1_ref, wc2_ref, bc2_ref,
             wn1a_ref, wn1b_ref, bn1_ref, wn2_ref, bn2_ref,
             lng_ref, lnb_ref, cs_ref,
             node_ref, space_ref):
    f32 = jnp.float32
    tab = tab_ref[...]
    feats = tab[:, :, 0:_D].reshape(_NBL, _D)
    space = tab[:, :, _D:_D + 3]
    gat = gat_ref[...].reshape(_EB, _TW)
    space_j = gat[:, _D:_D + 3]

    h_i = jnp.dot(feats, we1f_ref[...], preferred_element_type=f32)
    h = jnp.broadcast_to(h_i.reshape(_B, 1, _BN, _H),
                         (_B, _K, _BN, _H)).reshape(_EB, _H)
    h = h + jnp.dot(gat, we1j_ref[...], preferred_element_type=f32)
    he = []
    for b in range(_B):
        for k in range(_K):
            he.append(lax.dot_general(
                edg_ref[k, :, b, :], we1e_ref[...],
                (((0,), (0,)), ((), ())), preferred_element_type=f32))
    h = h + jnp.concatenate(he, axis=0)

    vec = space_j - jnp.broadcast_to(space[:, None], (_B, _K, _BN, 3)).reshape(_EB, 3)
    vec_t = jnp.transpose(vec)
    eu_t = jnp.sum(vec_t * vec_t, axis=0, keepdims=True)
    th = eu_t * 0.125
    s3 = jnp.sin(th)
    c3 = jnp.cos(th)
    s2 = 2.0 * s3 * c3
    c2 = 1.0 - 2.0 * s3 * s3
    s1 = 2.0 * s2 * c2
    c1 = 1.0 - 2.0 * s2 * s2
    s0 = 2.0 * s1 * c1
    c0 = 1.0 - 2.0 * s1 * s1
    ft = jnp.concatenate([s0, s1, s2, s3, c0, c1, c2, c3, eu_t], axis=0)
    fe = jnp.transpose(ft)
    h = h + jnp.dot(fe, we1fe_ref[...], preferred_element_type=f32)
    h = _gelu(h + be1_ref[...])

    m = jnp.dot(h, we2_ref[...], preferred_element_type=f32) + be2_ref[...]
    g = jax.nn.sigmoid(jnp.sum(m * wg_ref[...], axis=-1, keepdims=True) + bg_ref[0, 0])
    m = m * g

    hc = _gelu(jnp.dot(m, wc1_ref[...], preferred_element_type=f32) + bc1_ref[...])
    vw = jnp.sum(hc * wc2_ref[...], axis=-1, keepdims=True) + bc2_ref[0, 0]
    vw_t = jnp.transpose(vw)
    rn_t = vw_t * cs_ref[0, 0] / jnp.maximum(jnp.sqrt(eu_t), 1e-8)
    contrib = jnp.transpose(vec_t * rn_t)
    space_ref[...] = (jnp.sum(contrib.reshape(_B, _K, _BN, 3), axis=1)
                      + space)

    m_i = jnp.sum(m.reshape(_B, _K, _BN, _D), axis=1).reshape(_NBL, _D)
    mu = jnp.mean(feats, axis=-1, keepdims=True)
    var = jnp.mean((feats - mu) ** 2, axis=-1, keepdims=True)
    normed = (feats - mu) / jnp.sqrt(var + 1e-5) * lng_ref[...] + lnb_ref[...]
    hn = _gelu(jnp.dot(normed, wn1a_ref[...], preferred_element_type=f32)
               + jnp.dot(m_i, wn1b_ref[...], preferred_element_type=f32)
               + bn1_ref[...])
    node = (jnp.dot(hn, wn2_ref[...], preferred_element_type=f32)
            + bn2_ref[...] + feats)
    node_ref[...] = node.reshape(_B, _BN, _D)


def kernel(feats, space, edges, E_idx, W_e1, b_e1, W_e2, b_e2, W_g, b_g,
           W_c1, b_c1, W_c2, b_c2, W_n1, b_n1, W_n2, b_n2, ln_g, ln_b,
           coors_scale):
    f32 = jnp.float32
    feats = feats.astype(f32)
    space = space.astype(f32)
    table = jnp.concatenate(
        [feats, space, jnp.zeros((_B, _N, _TW - _D - 3), f32)], axis=-1)
    table_flat = table.reshape(_B * _N, _TW)
    idx = (jnp.transpose(E_idx.astype(jnp.int32), (0, 2, 1))
           + (jnp.arange(_B, dtype=jnp.int32) * _N)[:, None, None])
    edges_t = jnp.transpose(edges.astype(f32), (2, 3, 0, 1))

    W_e1 = W_e1.astype(f32)
    we1f = W_e1[0:_D]
    we1j = jnp.concatenate(
        [W_e1[_D:2 * _D], jnp.zeros((_TW - _D, _H), f32)], axis=0)
    we1fe = W_e1[2 * _D:2 * _D + 9]
    we1e = W_e1[2 * _D + 9:]
    row = lambda v: v.astype(f32).reshape(1, -1)
    wn1a = W_n1.astype(f32)[0:_D]
    wn1b = W_n1.astype(f32)[_D:]

    full = lambda a: pl.BlockSpec(a.shape, lambda i: (0,) * a.ndim)

    weights = [we1f, we1j, we1fe, we1e, row(b_e1),
               W_e2.astype(f32), row(b_e2), row(W_g), row(b_g).reshape(1, 1),
               W_c1.astype(f32), row(b_c1), row(W_c2), row(b_c2).reshape(1, 1),
               wn1a, wn1b, row(b_n1), W_n2.astype(f32), row(b_n2),
               row(ln_g), row(ln_b), row(coors_scale).reshape(1, 1)]

    nh = _N // _NS
    nodes, spaces = [], []
    for s in range(_NS):
        idx_s = idx[:, :, s * nh:(s + 1) * nh].reshape(_B * _K * nh)
        gat_s = _sc_gather(table_flat, idx_s).reshape(_B, _K, nh, _TW)
        off = s * (nh // _BN)
        node_s, space_s = pl.pallas_call(
            _tc_body,
            grid=(nh // _BN,),
            in_specs=[
                pl.BlockSpec((_B, _BN, _TW), lambda i, o=off: (0, i + o, 0)),
                pl.BlockSpec((_B, _K, _BN, _TW), lambda i: (0, 0, i, 0)),
                pl.BlockSpec((_K, _DE, _B, _BN), lambda i, o=off: (0, 0, 0, i + o)),
            ] + [full(w) for w in weights],
            out_specs=[
                pl.BlockSpec((_B, _BN, _D), lambda i: (0, i, 0)),
                pl.BlockSpec((_B, _BN, 3), lambda i: (0, i, 0)),
            ],
            out_shape=[
                jax.ShapeDtypeStruct((_B, nh, _D), f32),
                jax.ShapeDtypeStruct((_B, nh, 3), f32),
            ],
            compiler_params=pltpu.CompilerParams(
                dimension_semantics=("parallel",)),
        )(table, gat_s, edges_t, *weights)
        nodes.append(node_s)
        spaces.append(space_s)

    return (jnp.concatenate(nodes, axis=1), jnp.concatenate(spaces, axis=1))


# speedup vs baseline: 2.0582x; 1.1442x over previous
import functools

import jax
import jax.numpy as jnp
from jax import lax
from jax.experimental import pallas as pl
from jax.experimental.pallas import tpu as pltpu
from jax.experimental.pallas import tpu_sc as plsc

_B, _N, _K, _D = 2, 2048, 30, 32
_DE = 95
_H = 4 * _D
_TW = 128
_BN = 128
_NS = 2
_EB = _B * _K * _BN
_NBL = _B * _BN
_CH = 256


def _pick_chunk(per_w):
    for ch in range(min(per_w, 256), 7, -8):
        if per_w % ch == 0:
            return ch
    return per_w


def _sc_gather(table, idx):
    E = idx.shape[0]
    info = plsc.get_sparse_core_info()
    nw = info.num_cores * info.num_subcores
    per_w = E // nw
    _CH = _pick_chunk(per_w)
    n_ch = per_w // _CH
    mesh = plsc.VectorSubcoreMesh(core_axis_name="c", subcore_axis_name="s")

    @functools.partial(
        pl.kernel,
        mesh=mesh,
        out_type=jax.ShapeDtypeStruct((E, _TW), jnp.float32),
        scratch_types=[
            pltpu.VMEM((_CH,), jnp.int32),
            pltpu.VMEM((_CH,), jnp.int32),
            pltpu.VMEM((_CH, _TW), jnp.float32),
            pltpu.VMEM((_CH, _TW), jnp.float32),
            pltpu.SemaphoreType.DMA,
            pltpu.SemaphoreType.DMA,
            pltpu.SemaphoreType.DMA,
        ],
    )
    def gk(tab_hbm, idx_hbm, out_hbm, idx0, idx1, rows0, rows1,
           gsem, osem0, osem1):
        wid = lax.axis_index("s") * info.num_cores + lax.axis_index("c")
        base0 = wid * per_w
        idxs, rows, osems = (idx0, idx1), (rows0, rows1), (osem0, osem1)
        out_pending = [None, None]
        pltpu.sync_copy(idx_hbm.at[pl.ds(base0, _CH)], idx0)
        hg = pltpu.async_copy(tab_hbm.at[idx0], rows0, gsem)
        for c in range(1, n_ch + 1):
            p, q = c % 2, (c - 1) % 2
            if c < n_ch:
                pltpu.sync_copy(idx_hbm.at[pl.ds(base0 + c * _CH, _CH)],
                                idxs[p])
            hg.wait()
            out_pending[q] = pltpu.async_copy(
                rows[q], out_hbm.at[pl.ds(base0 + (c - 1) * _CH, _CH)],
                osems[q])
            if c < n_ch:
                if out_pending[p] is not None:
                    out_pending[p].wait()
                    out_pending[p] = None
                hg = pltpu.async_copy(tab_hbm.at[idxs[p]], rows[p], gsem)
        for h in out_pending:
            if h is not None:
                h.wait()

    return gk(table, idx)


def _gelu(x):
    return x * 0.5 * (1.0 + lax.erf(x * 0.7071067811865476))


def _tc_body(tab_ref, gat_ref, edg_ref,
             we1f_ref, we1j_ref, we1fe_ref, we1e_ref, be1_ref,
             we2_ref, be2_ref,
             wc1_ref, bc1_ref, wc2_ref, bc2_ref,
             wn1a_ref, wn1b_ref, bn1_ref, wn2_ref, bn2_ref,
             lng_ref, lnb_ref, cs_ref,
             node_ref, space_ref):
    f32 = jnp.float32
    tab = tab_ref[...]
    feats = tab[:, :, 0:_D].reshape(_NBL, _D)
    space = tab[:, :, _D:_D + 3]
    gat = gat_ref[...].reshape(_EB, _TW)
    space_j = gat[:, _D:_D + 3]

    h_i = jnp.dot(feats, we1f_ref[...], preferred_element_type=f32)
    h = jnp.broadcast_to(h_i.reshape(_B, 1, _BN, _H),
                         (_B, _K, _BN, _H)).reshape(_EB, _H)
    h = h + jnp.dot(gat, we1j_ref[...], preferred_element_type=f32)
    he = []
    for b in range(_B):
        for k in range(_K):
            he.append(lax.dot_general(
                edg_ref[k, :, b, :], we1e_ref[...],
                (((0,), (0,)), ((), ())), preferred_element_type=f32))
    h = h + jnp.concatenate(he, axis=0)

    vec = space_j - jnp.broadcast_to(space[:, None], (_B, _K, _BN, 3)).reshape(_EB, 3)
    vec_t = jnp.transpose(vec)
    eu_t = jnp.sum(vec_t * vec_t, axis=0, keepdims=True)
    th = eu_t * 0.125
    s3 = jnp.sin(th)
    c3 = jnp.cos(th)
    s2 = 2.0 * s3 * c3
    c2 = 1.0 - 2.0 * s3 * s3
    s1 = 2.0 * s2 * c2
    c1 = 1.0 - 2.0 * s2 * s2
    s0 = 2.0 * s1 * c1
    c0 = 1.0 - 2.0 * s1 * s1
    ft = jnp.concatenate([s0, s1, s2, s3, c0, c1, c2, c3, eu_t], axis=0)
    fe = jnp.transpose(ft)
    h = h + jnp.dot(fe, we1fe_ref[...], preferred_element_type=f32)
    h = _gelu(h + be1_ref[...])

    m_ext = jnp.dot(h, we2_ref[...], preferred_element_type=f32) + be2_ref[...]
    g = jax.nn.sigmoid(m_ext[:, _D:_D + 1])
    m = m_ext[:, 0:_D] * g

    hc = _gelu(jnp.dot(m, wc1_ref[...], preferred_element_type=f32) + bc1_ref[...])
    vw_t = lax.dot_general(wc2_ref[...], hc, (((1,), (1,)), ((), ())),
                           preferred_element_type=f32) + bc2_ref[0, 0]
    rn_t = vw_t * cs_ref[0, 0] / jnp.maximum(jnp.sqrt(eu_t), 1e-8)
    contrib = jnp.transpose(vec_t * rn_t)
    space_ref[...] = (jnp.sum(contrib.reshape(_B, _K, _BN, 3), axis=1)
                      + space)

    m_i = jnp.sum(m.reshape(_B, _K, _BN, _D), axis=1).reshape(_NBL, _D)
    mu = jnp.mean(feats, axis=-1, keepdims=True)
    var = jnp.mean((feats - mu) ** 2, axis=-1, keepdims=True)
    normed = (feats - mu) / jnp.sqrt(var + 1e-5) * lng_ref[...] + lnb_ref[...]
    hn = _gelu(jnp.dot(normed, wn1a_ref[...], preferred_element_type=f32)
               + jnp.dot(m_i, wn1b_ref[...], preferred_element_type=f32)
               + bn1_ref[...])
    node = (jnp.dot(hn, wn2_ref[...], preferred_element_type=f32)
            + bn2_ref[...] + feats)
    node_ref[...] = node.reshape(_B, _BN, _D)


def kernel(feats, space, edges, E_idx, W_e1, b_e1, W_e2, b_e2, W_g, b_g,
           W_c1, b_c1, W_c2, b_c2, W_n1, b_n1, W_n2, b_n2, ln_g, ln_b,
           coors_scale):
    f32 = jnp.float32
    feats = feats.astype(f32)
    space = space.astype(f32)
    table = jnp.concatenate(
        [feats, space, jnp.zeros((_B, _N, _TW - _D - 3), f32)], axis=-1)
    table_flat = table.reshape(_B * _N, _TW)
    idx = (jnp.transpose(E_idx.astype(jnp.int32), (0, 2, 1))
           + (jnp.arange(_B, dtype=jnp.int32) * _N)[:, None, None])
    edges_t = jnp.transpose(edges.astype(f32), (2, 3, 0, 1))

    W_e1 = W_e1.astype(f32)
    we1f = W_e1[0:_D]
    we1j = jnp.concatenate(
        [W_e1[_D:2 * _D], jnp.zeros((_TW - _D, _H), f32)], axis=0)
    we1fe = W_e1[2 * _D:2 * _D + 9]
    we1e = W_e1[2 * _D + 9:]
    row = lambda v: v.astype(f32).reshape(1, -1)
    wn1a = W_n1.astype(f32)[0:_D]
    wn1b = W_n1.astype(f32)[_D:]

    full = lambda a: pl.BlockSpec(a.shape, lambda i: (0,) * a.ndim)

    W_e2 = W_e2.astype(f32)
    W_g = W_g.astype(f32)
    we2x = jnp.concatenate([W_e2, W_e2 @ W_g], axis=1)
    be2x = jnp.concatenate(
        [b_e2.astype(f32), b_e2.astype(f32) @ W_g + b_g.astype(f32)]
    ).reshape(1, _D + 1)

    weights = [we1f, we1j, we1fe, we1e, row(b_e1),
               we2x, be2x,
               W_c1.astype(f32), row(b_c1), row(W_c2), row(b_c2).reshape(1, 1),
               wn1a, wn1b, row(b_n1), W_n2.astype(f32), row(b_n2),
               row(ln_g), row(ln_b), row(coors_scale).reshape(1, 1)]

    nh = _N // _NS
    nodes, spaces = [], []
    for s in range(_NS):
        idx_s = idx[:, :, s * nh:(s + 1) * nh].reshape(_B * _K * nh)
        gat_s = _sc_gather(table_flat, idx_s).reshape(_B, _K, nh, _TW)
        off = s * (nh // _BN)
        node_s, space_s = pl.pallas_call(
            _tc_body,
            grid=(nh // _BN,),
            in_specs=[
                pl.BlockSpec((_B, _BN, _TW), lambda i, o=off: (0, i + o, 0)),
                pl.BlockSpec((_B, _K, _BN, _TW), lambda i: (0, 0, i, 0)),
                pl.BlockSpec((_K, _DE, _B, _BN), lambda i, o=off: (0, 0, 0, i + o)),
            ] + [full(w) for w in weights],
            out_specs=[
                pl.BlockSpec((_B, _BN, _D), lambda i: (0, i, 0)),
                pl.BlockSpec((_B, _BN, 3), lambda i: (0, i, 0)),
            ],
            out_shape=[
                jax.ShapeDtypeStruct((_B, nh, _D), f32),
                jax.ShapeDtypeStruct((_B, nh, 3), f32),
            ],
            compiler_params=pltpu.CompilerParams(
                dimension_semantics=("parallel",)),
        )(table, gat_s, edges_t, *weights)
        nodes.append(node_s)
        spaces.append(space_s)

    return (jnp.concatenate(nodes, axis=1), jnp.concatenate(spaces, axis=1))
